# Initial kernel scaffold; baseline (speedup 1.0000x reference)
import functools

import jax
import jax.numpy as jnp
from jax import lax
from jax.experimental import pallas as pl
from jax.experimental.pallas import tpu as pltpu
from jax.experimental.pallas import tpu_sc as plsc

N = 10000
D = 128
E = 320000

NC = 2
NS = 16
NW = NC * NS
LANES = 16
BATCH = 128
NB = -(-E // (NW * BATCH))
EPAD = NW * NB * BATCH
NPAD = 10016
RPS = NPAD // NS

_mesh = plsc.VectorSubcoreMesh(
    core_axis_name="c", subcore_axis_name="s", num_cores=NC, num_subcores=NS)


def _pad_edges(edge_index):
    pad = EPAD - E
    src = jnp.concatenate([edge_index[0], jnp.zeros((pad,), jnp.int32)])
    dst = jnp.concatenate([edge_index[1], jnp.full((pad,), N, jnp.int32)])
    return (src.reshape(NC, NS, NB, BATCH), dst.reshape(NC, NS, NB, BATCH))


def _segsum_body(h_hbm, src_hbm, dst_hbm, out_s, out_d,
                 srcv, dstv, rowbuf, zdeg, onesb, accs, accd, gsem):
    cid = lax.axis_index("c")
    sid = lax.axis_index("s")

    zero16 = jnp.zeros((LANES,), jnp.float32)
    one16 = jnp.ones((LANES,), jnp.float32)

    def _zrow(i, _):
        for k in range(D // LANES):
            rowbuf[i, pl.ds(k * LANES, LANES)] = zero16
        zdeg[i, :] = zero16
        onesb[i, :] = one16
        return 0
    lax.fori_loop(0, BATCH, _zrow, 0)

    base = sid * RPS
    for t in range(RPS // BATCH):
        pltpu.sync_copy(rowbuf, accs.at[pl.ds(base + t * BATCH, BATCH)])
        pltpu.sync_copy(zdeg, accd.at[pl.ds(base + t * BATCH, BATCH)])
    rem = RPS % BATCH
    if rem:
        off = base + (RPS // BATCH) * BATCH
        pltpu.sync_copy(rowbuf.at[pl.ds(0, rem)], accs.at[pl.ds(off, rem)])
        pltpu.sync_copy(zdeg.at[pl.ds(0, rem)], accd.at[pl.ds(off, rem)])

    pltpu.sync_copy(src_hbm.at[cid, sid], srcv)
    pltpu.sync_copy(dst_hbm.at[cid, sid], dstv)

    plsc.subcore_barrier()

    def _batch(j, _):
        pltpu.async_copy(h_hbm.at[srcv.at[j]], rowbuf, gsem).wait()
        pltpu.sync_copy(rowbuf, accs.at[dstv.at[j]], add=True)
        pltpu.sync_copy(onesb, accd.at[dstv.at[j]], add=True)
        return 0
    lax.fori_loop(0, NB, _batch, 0)

    plsc.subcore_barrier()

    for t in range(RPS // BATCH):
        pltpu.sync_copy(accs.at[pl.ds(base + t * BATCH, BATCH)],
                        out_s.at[cid, pl.ds(base + t * BATCH, BATCH)])
    if rem:
        off = base + (RPS // BATCH) * BATCH
        pltpu.sync_copy(accs.at[pl.ds(off, rem)], out_s.at[cid, pl.ds(off, rem)])
    pltpu.sync_copy(accd.at[pl.ds(base, RPS)], out_d.at[cid, pl.ds(base, RPS)])


_segsum = functools.partial(
    pl.kernel,
    _segsum_body,
    out_type=(
        jax.ShapeDtypeStruct((NC, NPAD, D), jnp.float32),
        jax.ShapeDtypeStruct((NC, NPAD, LANES), jnp.float32),
    ),
    mesh=_mesh,
    scratch_types=[
        pltpu.VMEM((NB, BATCH), jnp.int32),
        pltpu.VMEM((NB, BATCH), jnp.int32),
        pltpu.VMEM((BATCH, D), jnp.float32),
        pltpu.VMEM((BATCH, LANES), jnp.float32),
        pltpu.VMEM((BATCH, LANES), jnp.float32),
        pltpu.VMEM_SHARED((NPAD, D), jnp.float32),
        pltpu.VMEM_SHARED((NPAD, LANES), jnp.float32),
        pltpu.SemaphoreType.DMA,
    ],
)()


_BLK = 400


def _layer_tc_body(s_ref, d_ref, w_ref, b_ref, o_ref):
    s = s_ref[0] + s_ref[1]
    d = d_ref[0] + d_ref[1]
    deg = jnp.maximum(d[:, 0:1], 1.0)
    agg = s / deg
    y = jnp.dot(agg, w_ref[...], preferred_element_type=jnp.float32) + b_ref[...]
    o_ref[...] = jnp.maximum(y, 0.0)


def _layer_tc(s_part, d_part, W, b):
    return pl.pallas_call(
        _layer_tc_body,
        grid=(N // _BLK,),
        in_specs=[
            pl.BlockSpec((NC, _BLK, D), lambda i: (0, i, 0)),
            pl.BlockSpec((NC, _BLK, LANES), lambda i: (0, i, 0)),
            pl.BlockSpec((D, D), lambda i: (0, 0)),
            pl.BlockSpec((1, D), lambda i: (0, 0)),
        ],
        out_specs=pl.BlockSpec((_BLK, D), lambda i: (i, 0)),
        out_shape=jax.ShapeDtypeStruct((N, D), jnp.float32),
    )(s_part, d_part, W, b.reshape(1, D))


def _score_body(h_hbm, u_hbm, v_hbm, out, uidx, vidx, ubuf, vbuf, sbuf, usem, vsem):
    cid = lax.axis_index("c")
    sid = lax.axis_index("s")
    pltpu.sync_copy(u_hbm.at[cid, sid], uidx)
    pltpu.sync_copy(v_hbm.at[cid, sid], vidx)

    def _batch(j, _):
        cu = pltpu.async_copy(h_hbm.at[uidx.at[j]], ubuf, usem)
        cv = pltpu.async_copy(h_hbm.at[vidx.at[j]], vbuf, vsem)
        cu.wait()
        cv.wait()

        def _edge(r, _):
            acc = ubuf[r, pl.ds(0, LANES)] * vbuf[r, pl.ds(0, LANES)]
            for k in range(1, D // LANES):
                acc = acc + (ubuf[r, pl.ds(k * LANES, LANES)]
                             * vbuf[r, pl.ds(k * LANES, LANES)])
            sbuf[r] = jnp.sum(acc)
            return 0
        lax.fori_loop(0, BATCH, _edge, 0)
        pltpu.sync_copy(sbuf, out.at[cid, sid, j])
        return 0
    lax.fori_loop(0, NB, _batch, 0)


_score = functools.partial(
    pl.kernel,
    _score_body,
    out_type=jax.ShapeDtypeStruct((NC, NS, NB, BATCH), jnp.float32),
    mesh=_mesh,
    scratch_types=[
        pltpu.VMEM((NB, BATCH), jnp.int32),
        pltpu.VMEM((NB, BATCH), jnp.int32),
        pltpu.VMEM((BATCH, D), jnp.float32),
        pltpu.VMEM((BATCH, D), jnp.float32),
        pltpu.VMEM((BATCH,), jnp.float32),
        pltpu.SemaphoreType.DMA,
        pltpu.SemaphoreType.DMA,
    ],
)()


def kernel(x, block0_edge_index, block1_edge_index, block2_edge_index,
           pos_edge_index, neg_edge_index, W1, b1, W2, b2, W3, b3):
    h = x
    for ei, W, b in ((block0_edge_index, W1, b1),
                     (block1_edge_index, W2, b2),
                     (block2_edge_index, W3, b3)):
        src, dst = _pad_edges(ei)
        s_part, d_part = _segsum(h, src, dst)
        h = _layer_tc(s_part, d_part, W, b)

    pu, pv = _pad_edges(pos_edge_index)
    nu, nv = _pad_edges(neg_edge_index)
    pos = _score(h, pu, pv).reshape(EPAD)[:E].reshape(E, 1)
    neg = _score(h, nu, nv).reshape(EPAD)[:E].reshape(E, 1)
    return (pos, neg)


# speedup vs baseline: 2.1661x; 2.1661x over previous
import jax
import jax.numpy as jnp
from jax import lax
from jax.experimental import pallas as pl
from jax.experimental.pallas import tpu as pltpu
from jax.experimental.pallas import tpu_sc as plsc

N = 10000
D = 128
E = 320000

NC = 2
NS = 16
NW = NC * NS
LANES = 16
BATCH = 128
NB = -(-E // (NW * BATCH))
EPAD = NW * NB * BATCH
NPAD = 10112
RPS = NPAD // NS
CHUNK_OFFS = (0, 128, 256, 384, RPS - BATCH)
RCH = len(CHUNK_OFFS)


def _sc_mesh():
    return plsc.VectorSubcoreMesh(
        core_axis_name="c", subcore_axis_name="s", num_cores=NC, num_subcores=NS)


def _pad_edges(edge_index, dst_pad=N):
    pad = EPAD - E
    src = jnp.concatenate([edge_index[0], jnp.zeros((pad,), jnp.int32)])
    dst = jnp.concatenate([edge_index[1], jnp.full((pad,), dst_pad, jnp.int32)])
    return (src.reshape(NC, NS, NB, BATCH), dst.reshape(NC, NS, NB, BATCH))


def _segsum_body(h_hbm, src_hbm, dst_hbm, z_hbm, out_s,
                 srcv, dstidx, rowbuf, accs, gsem):
    cid = lax.axis_index("c")
    sid = lax.axis_index("s")

    pltpu.sync_copy(src_hbm.at[cid, sid], srcv)

    @pl.when(sid == 0)
    def _():
        pltpu.sync_copy(z_hbm, accs)
    plsc.subcore_barrier()

    def _batch(j, _):
        pltpu.async_copy(h_hbm.at[srcv.at[j]], rowbuf, gsem).wait()
        pltpu.sync_copy(dst_hbm.at[cid, sid, j], dstidx)
        pltpu.sync_copy(rowbuf, accs.at[dstidx], add=True)
        return 0
    lax.fori_loop(0, NB, _batch, 0)

    plsc.subcore_barrier()

    @pl.when(sid == 0)
    def _():
        pltpu.sync_copy(accs, out_s.at[cid])


def _make_segsum():
    return pl.kernel(
        _segsum_body,
        out_type=jax.ShapeDtypeStruct((NC, NPAD, D), jnp.float32),
        mesh=_sc_mesh(),
        scratch_types=[
            pltpu.VMEM((NB, BATCH), jnp.int32),
            pltpu.VMEM((BATCH,), jnp.int32),
            pltpu.VMEM((BATCH, D), jnp.float32),
            pltpu.VMEM_SHARED((NPAD, D), jnp.float32),
            pltpu.SemaphoreType.DMA,
        ],
    )


_BLK = 400


def _layer_tc_body(s_ref, d_ref, w_ref, b_ref, o_ref):
    s = s_ref[0] + s_ref[1]
    d = d_ref[0] + d_ref[1]
    deg = jnp.maximum(d[:, 0:1], 1.0)
    agg = s / deg
    y = jnp.dot(agg, w_ref[...], preferred_element_type=jnp.float32) + b_ref[...]
    o_ref[...] = jnp.maximum(y, 0.0)


def _layer_tc(s_part, d_part, W, b):
    return pl.pallas_call(
        _layer_tc_body,
        grid=(N // _BLK,),
        in_specs=[
            pl.BlockSpec((NC, _BLK, D), lambda i: (0, i, 0)),
            pl.BlockSpec((NC, _BLK, D), lambda i: (0, i, 0)),
            pl.BlockSpec((D, D), lambda i: (0, 0)),
            pl.BlockSpec((1, D), lambda i: (0, 0)),
        ],
        out_specs=pl.BlockSpec((_BLK, D), lambda i: (i, 0)),
        out_shape=jax.ShapeDtypeStruct((N, D), jnp.float32),
    )(s_part, d_part, W, b.reshape(1, D))


def _score_body(h_hbm, u_hbm, v_hbm, out, uidx, vidx, ubuf, vbuf, pbuf,
                usem, vsem):
    cid = lax.axis_index("c")
    sid = lax.axis_index("s")
    pltpu.sync_copy(u_hbm.at[cid, sid], uidx)
    pltpu.sync_copy(v_hbm.at[cid, sid], vidx)
    tile_base = (cid * NS + sid) * (NB * BATCH)

    def _batch(j, _):
        cu = pltpu.async_copy(h_hbm.at[uidx.at[j]], ubuf, usem)
        cv = pltpu.async_copy(h_hbm.at[vidx.at[j]], vbuf, vsem)
        cu.wait()
        cv.wait()

        def _edge(r, _):
            acc = ubuf[r, pl.ds(0, LANES)] * vbuf[r, pl.ds(0, LANES)]
            for k in range(1, D // LANES):
                acc = acc + (ubuf[r, pl.ds(k * LANES, LANES)]
                             * vbuf[r, pl.ds(k * LANES, LANES)])
            pbuf[r, :] = acc
            return 0
        lax.fori_loop(0, BATCH, _edge, 0)
        pltpu.sync_copy(pbuf, out.at[pl.ds(tile_base + j * BATCH, BATCH)])
        return 0
    lax.fori_loop(0, NB, _batch, 0)


def _make_score():
    return pl.kernel(
        _score_body,
        out_type=jax.ShapeDtypeStruct((EPAD, LANES), jnp.float32),
        mesh=_sc_mesh(),
        scratch_types=[
            pltpu.VMEM((NB, BATCH), jnp.int32),
            pltpu.VMEM((NB, BATCH), jnp.int32),
            pltpu.VMEM((BATCH, D), jnp.float32),
            pltpu.VMEM((BATCH, D), jnp.float32),
            pltpu.VMEM((BATCH, LANES), jnp.float32),
            pltpu.SemaphoreType.DMA,
            pltpu.SemaphoreType.DMA,
        ],
    )


_RBLK = 5056


def _reduce_tc_body(p_ref, m_ref, o_ref):
    o_ref[...] = jnp.dot(p_ref[...], m_ref[...],
                         preferred_element_type=jnp.float32)


def _reduce_tc(p, m):
    rows = EPAD // 8
    p2 = p.reshape(rows, 128)
    return pl.pallas_call(
        _reduce_tc_body,
        grid=(rows // _RBLK,),
        in_specs=[
            pl.BlockSpec((_RBLK, 128), lambda i: (i, 0)),
            pl.BlockSpec((128, 8), lambda i: (0, 0)),
        ],
        out_specs=pl.BlockSpec((_RBLK, 8), lambda i: (i, 0)),
        out_shape=jax.ShapeDtypeStruct((rows, 8), jnp.float32),
    )(p2, m).reshape(EPAD)


def kernel(x, block0_edge_index, block1_edge_index, block2_edge_index,
           pos_edge_index, neg_edge_index, W1, b1, W2, b2, W3, b3):
    segsum = _make_segsum()
    score = _make_score()

    e0, e1, e2 = (_pad_edges(block0_edge_index), _pad_edges(block1_edge_index),
                  _pad_edges(block2_edge_index))
    z128 = jnp.zeros((NPAD, D), jnp.float32)
    ones = jnp.ones((N, D), jnp.float32)
    d0 = segsum(ones, e0[0], e0[1], z128)
    d1 = segsum(ones, e1[0], e1[1], z128)
    d2 = segsum(ones, e2[0], e2[1], z128)

    h = x
    for (src, dst), dp, W, b in ((e0, d0, W1, b1), (e1, d1, W2, b2),
                                 (e2, d2, W3, b3)):
        s_part = segsum(h, src, dst, z128)
        h = _layer_tc(s_part, dp, W, b)

    pu, pv = _pad_edges(pos_edge_index, dst_pad=0)
    nu, nv = _pad_edges(neg_edge_index, dst_pad=0)
    m = (jnp.arange(128, dtype=jnp.int32)[:, None] // LANES
         == jnp.arange(8, dtype=jnp.int32)[None, :]).astype(jnp.float32)
    pos = _reduce_tc(score(h, pu, pv), m)[:E].reshape(E, 1)
    neg = _reduce_tc(score(h, nu, nv), m)[:E].reshape(E, 1)
    return (pos, neg)
